# Initial kernel scaffold; baseline (speedup 1.0000x reference)
import functools

import jax
import jax.numpy as jnp
from jax import lax
from jax.experimental import pallas as pl
from jax.experimental.pallas import tpu as pltpu
from jax.experimental.pallas import tpu_sc as plsc

_NN = 100000
_NE = 6400000
_NH = 8
_ROWS = _NE // 128
_NC, _NS = 2, 16
_NW = _NC * _NS
_CH_ROWS = 8
_CH_E = _CH_ROWS * 128
_TOT_CH = _ROWS // _CH_ROWS
_BASE_CH = _TOT_CH // _NW
_EXTRA = _TOT_CH - _BASE_CH * _NW
_DRAIN = _NN // _NS

_LN2 = 0.6931471805599453
_C3, _C5, _C7 = 1.0 / 3.0, 1.0 / 5.0, 1.0 / 7.0


def _sc_body(inp_hbm, idx_hbm, zero_hbm, out_hbm, acc, vin, vidx, vxl):
    cid = lax.axis_index("c")
    sid = lax.axis_index("s")
    wid = sid * _NC + cid

    pltpu.sync_copy(zero_hbm, acc.at[pl.ds(sid * _DRAIN, _DRAIN), :])
    plsc.subcore_barrier()

    nch = _BASE_CH + jnp.where(wid < _EXTRA, 1, 0)
    start = wid * _BASE_CH + jnp.minimum(wid, _EXTRA)

    rowoff = lax.iota(jnp.int32, 16) >> 3
    coloff = lax.iota(jnp.int32, 16) & 7

    def chunk_body(g, carry):
        ch = start + g
        pltpu.sync_copy(inp_hbm.at[pl.ds(ch * (_CH_E * _NH), _CH_E * _NH)], vin)
        pltpu.sync_copy(idx_hbm.at[pl.ds(ch * _CH_ROWS, _CH_ROWS), :], vidx)

        def grp(i, c2):
            x = vin[pl.ds(i * 16, 16)]
            b = plsc.bitcast(x, jnp.int32)
            e = (127 - (b >> 23)).astype(jnp.float32)
            m = plsc.bitcast((b & 0x7FFFFF) | 0x3F800000, jnp.float32)
            t = (1.0 - m) / (1.0 + m)
            t2 = t * t
            p = _C3 + t2 * (_C5 + t2 * _C7)
            nlog = e * _LN2 + (2.0 * t + 2.0 * t * t2 * p)
            val = x * nlog
            plsc.store_scatter(vxl, [2 * i + rowoff, coloff], val)
            return c2

        lax.fori_loop(0, _CH_E * _NH // 16, grp, 0)

        def srow(j, c3):
            pltpu.sync_copy(vxl.at[pl.ds(j * 128, 128), :],
                            acc.at[vidx.at[j]], add=True)
            return c3

        lax.fori_loop(0, _CH_ROWS, srow, 0)
        return carry

    lax.fori_loop(0, nch, chunk_body, 0)

    plsc.subcore_barrier()
    pltpu.sync_copy(acc.at[pl.ds(sid * _DRAIN, _DRAIN), :],
                    out_hbm.at[cid, pl.ds(sid * _DRAIN, _DRAIN), :])


_sc_entropy = functools.partial(
    pl.kernel,
    out_type=jax.ShapeDtypeStruct((_NC, _NN, _NH), jnp.float32),
    mesh=plsc.VectorSubcoreMesh(core_axis_name="c", subcore_axis_name="s"),
    scratch_types=[
        pltpu.VMEM_SHARED((_NN, _NH), jnp.float32),
        pltpu.VMEM((_CH_E * _NH,), jnp.float32),
        pltpu.VMEM((_CH_ROWS, 128), jnp.int32),
        pltpu.VMEM((_CH_E, _NH), jnp.float32),
    ],
)(_sc_body)


def _combine_body(p_ref, o_ref):
    o_ref[...] = p_ref[0] + p_ref[1]


def kernel(input, edge_index):
    inp_flat = input.reshape(_NE * _NH)
    idx2d = edge_index.reshape(_ROWS, 128)
    zeros = jnp.zeros((_DRAIN, _NH), jnp.float32)
    partials = _sc_entropy(inp_flat, idx2d, zeros)
    p3 = partials.reshape(_NC, _NN * _NH // 128, 128)
    out = pl.pallas_call(
        _combine_body,
        out_shape=jax.ShapeDtypeStruct((_NN * _NH // 128, 128), jnp.float32),
    )(p3)
    return out.reshape(_NN, _NH)


# speedup vs baseline: 3.6831x; 3.6831x over previous
import functools

import jax
import jax.numpy as jnp
from jax import lax
from jax.experimental import pallas as pl
from jax.experimental.pallas import tpu as pltpu
from jax.experimental.pallas import tpu_sc as plsc

_NN = 100000
_NE = 6400000
_NH = 8
_ROWS = _NE // 128
_NC, _NS = 2, 16
_NW = _NC * _NS
_CH_ROWS = 8
_CH_E = _CH_ROWS * 128
_TOT_CH = _ROWS // _CH_ROWS
_BASE_CH = _TOT_CH // _NW
_EXTRA = _TOT_CH - _BASE_CH * _NW
_DRAIN = _NN // _NS

_LN2 = 0.6931471805599453
_C3, _C5, _C7 = 1.0 / 3.0, 1.0 / 5.0, 1.0 / 7.0


def _sc_body(inp_hbm, idx_hbm, zero_hbm, out_hbm, acc, vin, vidx, vxl):
    cid = lax.axis_index("c")
    sid = lax.axis_index("s")
    wid = sid * _NC + cid

    pltpu.sync_copy(zero_hbm, acc.at[pl.ds(sid * _DRAIN, _DRAIN), :])
    plsc.subcore_barrier()

    nch = _BASE_CH + jnp.where(wid < _EXTRA, 1, 0)
    start = wid * _BASE_CH + jnp.minimum(wid, _EXTRA)

    rowoff = lax.iota(jnp.int32, 16) >> 3
    coloff = lax.iota(jnp.int32, 16) & 7

    def chunk_body(g, carry):
        ch = start + g
        pltpu.sync_copy(inp_hbm.at[pl.ds(ch * (_CH_E * _NH), _CH_E * _NH)], vin)
        pltpu.sync_copy(idx_hbm.at[pl.ds(ch * _CH_ROWS, _CH_ROWS), :], vidx)

        def grp(i, c2):
            x = vin[pl.ds(i * 16, 16)]
            b = plsc.bitcast(x, jnp.int32)
            e = (127 - (b >> 23)).astype(jnp.float32)
            m = plsc.bitcast((b & 0x7FFFFF) | 0x3F800000, jnp.float32)
            t = (1.0 - m) / (1.0 + m)
            t2 = t * t
            p = _C3 + t2 * (_C5 + t2 * _C7)
            nlog = e * _LN2 + (2.0 * t + 2.0 * t * t2 * p)
            val = x * nlog
            plsc.store_scatter(vxl, [2 * i + rowoff, coloff], val)
            return c2

        lax.fori_loop(0, _CH_E * _NH // 16, grp, 0)

        def srow(j, c3):
            pltpu.sync_copy(vxl.at[pl.ds(j * 128, 128), :],
                            acc.at[vidx.at[j]], add=True)
            return c3

        lax.fori_loop(0, _CH_ROWS, srow, 0)
        return carry

    lax.fori_loop(0, nch, chunk_body, 0)

    plsc.subcore_barrier()
    pltpu.sync_copy(acc.at[pl.ds(sid * _DRAIN, _DRAIN), :],
                    out_hbm.at[cid, pl.ds(sid * _DRAIN, _DRAIN), :])


_sc_entropy = functools.partial(
    pl.kernel,
    out_type=jax.ShapeDtypeStruct((_NC, _NN, _NH), jnp.float32),
    mesh=plsc.VectorSubcoreMesh(core_axis_name="c", subcore_axis_name="s"),
    scratch_types=[
        pltpu.VMEM_SHARED((_NN, _NH), jnp.float32),
        pltpu.VMEM((_CH_E * _NH,), jnp.float32),
        pltpu.VMEM((_CH_ROWS, 128), jnp.int32),
        pltpu.VMEM((_CH_E, _NH), jnp.float32),
    ],
    compiler_params=pltpu.CompilerParams(use_tc_tiling_on_sc=False,
                                         needs_layout_passes=False),
)(_sc_body)


def _combine_body(p_ref, o_ref):
    o_ref[...] = p_ref[0] + p_ref[1]


def kernel(input, edge_index):
    inp_flat = input.reshape(_NE * _NH)
    idx2d = edge_index.reshape(_ROWS, 128)
    zeros = jnp.zeros((_DRAIN, _NH), jnp.float32)
    partials = _sc_entropy(inp_flat, idx2d, zeros)
    p3 = partials.reshape(_NC, _NN * _NH // 128, 128)
    out = pl.pallas_call(
        _combine_body,
        out_shape=jax.ShapeDtypeStruct((_NN * _NH // 128, 128), jnp.float32),
    )(p3)
    return out.reshape(_NN, _NH)


# speedup vs baseline: 3.8684x; 1.0503x over previous
import functools

import jax
import jax.numpy as jnp
from jax import lax
from jax.experimental import pallas as pl
from jax.experimental.pallas import tpu as pltpu
from jax.experimental.pallas import tpu_sc as plsc

_NN = 100000
_NE = 6400000
_NH = 8
_ROWS = _NE // 128
_NC, _NS = 2, 16
_NW = _NC * _NS
_CH_ROWS = 8
_CH_E = _CH_ROWS * 128
_CH_EL = _CH_E * _NH
_TOT_CH = _ROWS // _CH_ROWS
_BASE_CH = _TOT_CH // _NW
_EXTRA = _TOT_CH - _BASE_CH * _NW
_DRAIN = _NN // _NS

_LN2 = 0.6931471805599453
_C3, _C5, _C7 = 1.0 / 3.0, 1.0 / 5.0, 1.0 / 7.0


def _sc_body(inp_hbm, idx_hbm, zero_hbm, out_hbm,
             acc, vin0, vin1, vidx0, vidx1, vxl0, vxl1,
             in_sem0, in_sem1, scat_sem0, scat_sem1):
    cid = lax.axis_index("c")
    sid = lax.axis_index("s")
    wid = sid * _NC + cid

    bufs = ((vin0, vidx0, vxl0, in_sem0, scat_sem0),
            (vin1, vidx1, vxl1, in_sem1, scat_sem1))

    pltpu.sync_copy(zero_hbm, acc.at[pl.ds(sid * _DRAIN, _DRAIN), :])
    plsc.subcore_barrier()

    nch = _BASE_CH + jnp.where(wid < _EXTRA, 1, 0)
    start = wid * _BASE_CH + jnp.minimum(wid, _EXTRA)

    rowoff = lax.iota(jnp.int32, 16) >> 3
    coloff = lax.iota(jnp.int32, 16) & 7

    def issue_in(g, buf):
        vin, vidx, _, in_sem, _ = buf
        ch = start + g
        pltpu.async_copy(inp_hbm.at[pl.ds(ch * _CH_EL, _CH_EL)], vin, in_sem)
        pltpu.async_copy(idx_hbm.at[pl.ds(ch * _CH_ROWS, _CH_ROWS), :], vidx,
                         in_sem)

    def wait_in(g, buf):
        vin, vidx, _, in_sem, _ = buf
        ch = start + g
        pltpu.make_async_copy(inp_hbm.at[pl.ds(ch * _CH_EL, _CH_EL)], vin,
                              in_sem).wait()
        pltpu.make_async_copy(idx_hbm.at[pl.ds(ch * _CH_ROWS, _CH_ROWS), :],
                              vidx, in_sem).wait()

    def issue_scat(buf):
        _, vidx, vxl, _, scat_sem = buf

        def srow(j, c):
            pltpu.async_copy(vxl.at[pl.ds(j * 128, 128), :],
                             acc.at[vidx.at[j]], scat_sem, add=True)
            return c

        lax.fori_loop(0, _CH_ROWS, srow, 0)

    def wait_scat(buf):
        _, vidx, vxl, _, scat_sem = buf

        def srow(j, c):
            pltpu.make_async_copy(vxl.at[pl.ds(j * 128, 128), :],
                                  acc.at[vidx.at[j]], scat_sem).wait()
            return c

        lax.fori_loop(0, _CH_ROWS, srow, 0)

    def compute(buf):
        vin, _, vxl, _, _ = buf

        def grp(i, c2):
            x = vin[pl.ds(i * 16, 16)]
            b = plsc.bitcast(x, jnp.int32)
            e = (127 - (b >> 23)).astype(jnp.float32)
            m = plsc.bitcast((b & 0x7FFFFF) | 0x3F800000, jnp.float32)
            t = (1.0 - m) / (1.0 + m)
            t2 = t * t
            p = _C3 + t2 * (_C5 + t2 * _C7)
            nlog = e * _LN2 + (2.0 * t + 2.0 * t * t2 * p)
            val = x * nlog
            plsc.store_scatter(vxl, [2 * i + rowoff, coloff], val)
            return c2

        lax.fori_loop(0, _CH_EL // 16, grp, 0)

    def process(g, this, other):
        @pl.when(g < nch)
        def _():
            wait_in(g, this)
            compute(this)

            @pl.when(g >= 1)
            def _():
                wait_scat(other)

            @pl.when(g + 1 < nch)
            def _():
                issue_in(g + 1, other)

            issue_scat(this)

    issue_in(0, bufs[0])

    def pair(k, c):
        process(2 * k, bufs[0], bufs[1])
        process(2 * k + 1, bufs[1], bufs[0])
        return c

    lax.fori_loop(0, (_BASE_CH + 2) // 2, pair, 0)

    @pl.when(lax.rem(nch - 1, 2) == 0)
    def _():
        wait_scat(bufs[0])

    @pl.when(lax.rem(nch - 1, 2) == 1)
    def _():
        wait_scat(bufs[1])

    plsc.subcore_barrier()
    pltpu.sync_copy(acc.at[pl.ds(sid * _DRAIN, _DRAIN), :],
                    out_hbm.at[cid, pl.ds(sid * _DRAIN, _DRAIN), :])


_sc_entropy = functools.partial(
    pl.kernel,
    out_type=jax.ShapeDtypeStruct((_NC, _NN, _NH), jnp.float32),
    mesh=plsc.VectorSubcoreMesh(core_axis_name="c", subcore_axis_name="s"),
    scratch_types=[
        pltpu.VMEM_SHARED((_NN, _NH), jnp.float32),
        pltpu.VMEM((_CH_EL,), jnp.float32),
        pltpu.VMEM((_CH_EL,), jnp.float32),
        pltpu.VMEM((_CH_ROWS, 128), jnp.int32),
        pltpu.VMEM((_CH_ROWS, 128), jnp.int32),
        pltpu.VMEM((_CH_E, _NH), jnp.float32),
        pltpu.VMEM((_CH_E, _NH), jnp.float32),
        pltpu.SemaphoreType.DMA,
        pltpu.SemaphoreType.DMA,
        pltpu.SemaphoreType.DMA,
        pltpu.SemaphoreType.DMA,
    ],
    compiler_params=pltpu.CompilerParams(use_tc_tiling_on_sc=False,
                                         needs_layout_passes=False),
)(_sc_body)


def _combine_body(p_ref, o_ref):
    o_ref[...] = p_ref[0] + p_ref[1]


def kernel(input, edge_index):
    inp_flat = input.reshape(_NE * _NH)
    idx2d = edge_index.reshape(_ROWS, 128)
    zeros = jnp.zeros((_DRAIN, _NH), jnp.float32)
    partials = _sc_entropy(inp_flat, idx2d, zeros)
    p3 = partials.reshape(_NC, _NN * _NH // 128, 128)
    out = pl.pallas_call(
        _combine_body,
        out_shape=jax.ShapeDtypeStruct((_NN * _NH // 128, 128), jnp.float32),
    )(p3)
    return out.reshape(_NN, _NH)


# speedup vs baseline: 6.0966x; 1.5760x over previous
import functools

import jax
import jax.numpy as jnp
from jax import lax
from jax.experimental import pallas as pl
from jax.experimental.pallas import tpu as pltpu
from jax.experimental.pallas import tpu_sc as plsc

_NN = 100000
_NE = 6400000
_NH = 8
_ROWS = _NE // 128
_NC, _NS = 2, 16
_NW = _NC * _NS
_CH_ROWS = 8
_CH_E = _CH_ROWS * 128
_CH_EL = _CH_E * _NH
_TOT_CH = _ROWS // _CH_ROWS
_BASE_CH = _TOT_CH // _NW
_EXTRA = _TOT_CH - _BASE_CH * _NW
_DRAIN = _NN // _NS

_LN2 = 0.6931471805599453
_C3, _C5, _C7 = 1.0 / 3.0, 1.0 / 5.0, 1.0 / 7.0


def _sc_body(inp_hbm, idx_hbm, zero_hbm, out_hbm,
             acc, vin0, vin1, vidx0, vidx1, vxl0, vxl1,
             in_sem0, in_sem1, scat_sem0, scat_sem1):
    cid = lax.axis_index("c")
    sid = lax.axis_index("s")
    wid = sid * _NC + cid

    bufs = ((vin0, vidx0, vxl0, in_sem0, scat_sem0),
            (vin1, vidx1, vxl1, in_sem1, scat_sem1))

    pltpu.sync_copy(zero_hbm, acc.at[pl.ds(sid * _DRAIN, _DRAIN), :])
    plsc.subcore_barrier()

    nch = _BASE_CH + jnp.where(wid < _EXTRA, 1, 0)
    start = wid * _BASE_CH + jnp.minimum(wid, _EXTRA)

    rowoff = lax.iota(jnp.int32, 16) >> 3
    coloff = lax.iota(jnp.int32, 16) & 7

    def issue_in(g, buf):
        vin, vidx, _, in_sem, _ = buf
        ch = start + g
        pltpu.async_copy(inp_hbm.at[pl.ds(ch * _CH_EL, _CH_EL)], vin, in_sem)
        pltpu.async_copy(idx_hbm.at[pl.ds(ch * _CH_ROWS, _CH_ROWS), :], vidx,
                         in_sem)

    def wait_in(g, buf):
        vin, vidx, _, in_sem, _ = buf
        ch = start + g
        pltpu.make_async_copy(inp_hbm.at[pl.ds(ch * _CH_EL, _CH_EL)], vin,
                              in_sem).wait()
        pltpu.make_async_copy(idx_hbm.at[pl.ds(ch * _CH_ROWS, _CH_ROWS), :],
                              vidx, in_sem).wait()

    def issue_scat(buf):
        _, vidx, vxl, _, scat_sem = buf

        def srow(j, c):
            pltpu.async_copy(vxl.at[pl.ds(j * 128, 128), :],
                             acc.at[vidx.at[j]], scat_sem, add=True)
            return c

        lax.fori_loop(0, _CH_ROWS, srow, 0)

    def wait_scat(buf):
        _, vidx, vxl, _, scat_sem = buf

        def srow(j, c):
            pltpu.make_async_copy(vxl.at[pl.ds(j * 128, 128), :],
                                  acc.at[vidx.at[j]], scat_sem).wait()
            return c

        lax.fori_loop(0, _CH_ROWS, srow, 0)

    def compute(buf):
        vin, _, vxl, _, _ = buf

        @plsc.parallel_loop(0, _CH_EL // 16, unroll=8)
        def grp(i):
            x = vin[pl.ds(i * 16, 16)]
            b = plsc.bitcast(x, jnp.int32)
            e = (127 - (b >> 23)).astype(jnp.float32)
            m = plsc.bitcast((b & 0x7FFFFF) | 0x3F800000, jnp.float32)
            t = (1.0 - m) / (1.0 + m)
            t2 = t * t
            p = _C3 + t2 * (_C5 + t2 * _C7)
            nlog = e * _LN2 + (2.0 * t + 2.0 * t * t2 * p)
            val = x * nlog
            plsc.store_scatter(vxl, [2 * i + rowoff, coloff], val)

    def process(g, this, other):
        @pl.when(g < nch)
        def _():
            wait_in(g, this)
            compute(this)

            @pl.when(g >= 1)
            def _():
                wait_scat(other)

            @pl.when(g + 1 < nch)
            def _():
                issue_in(g + 1, other)

            issue_scat(this)

    issue_in(0, bufs[0])

    def pair(k, c):
        process(2 * k, bufs[0], bufs[1])
        process(2 * k + 1, bufs[1], bufs[0])
        return c

    lax.fori_loop(0, (_BASE_CH + 2) // 2, pair, 0)

    @pl.when(lax.rem(nch - 1, 2) == 0)
    def _():
        wait_scat(bufs[0])

    @pl.when(lax.rem(nch - 1, 2) == 1)
    def _():
        wait_scat(bufs[1])

    plsc.subcore_barrier()
    pltpu.sync_copy(acc.at[pl.ds(sid * _DRAIN, _DRAIN), :],
                    out_hbm.at[cid, pl.ds(sid * _DRAIN, _DRAIN), :])


_sc_entropy = functools.partial(
    pl.kernel,
    out_type=jax.ShapeDtypeStruct((_NC, _NN, _NH), jnp.float32),
    mesh=plsc.VectorSubcoreMesh(core_axis_name="c", subcore_axis_name="s"),
    scratch_types=[
        pltpu.VMEM_SHARED((_NN, _NH), jnp.float32),
        pltpu.VMEM((_CH_EL,), jnp.float32),
        pltpu.VMEM((_CH_EL,), jnp.float32),
        pltpu.VMEM((_CH_ROWS, 128), jnp.int32),
        pltpu.VMEM((_CH_ROWS, 128), jnp.int32),
        pltpu.VMEM((_CH_E, _NH), jnp.float32),
        pltpu.VMEM((_CH_E, _NH), jnp.float32),
        pltpu.SemaphoreType.DMA,
        pltpu.SemaphoreType.DMA,
        pltpu.SemaphoreType.DMA,
        pltpu.SemaphoreType.DMA,
    ],
    compiler_params=pltpu.CompilerParams(use_tc_tiling_on_sc=False,
                                         needs_layout_passes=False),
)(_sc_body)


def _combine_body(p_ref, o_ref):
    o_ref[...] = p_ref[0] + p_ref[1]


def kernel(input, edge_index):
    inp_flat = input.reshape(_NE * _NH)
    idx2d = edge_index.reshape(_ROWS, 128)
    zeros = jnp.zeros((_DRAIN, _NH), jnp.float32)
    partials = _sc_entropy(inp_flat, idx2d, zeros)
    p3 = partials.reshape(_NC, _NN * _NH // 128, 128)
    out = pl.pallas_call(
        _combine_body,
        out_shape=jax.ShapeDtypeStruct((_NN * _NH // 128, 128), jnp.float32),
    )(p3)
    return out.reshape(_NN, _NH)
